# Initial kernel scaffold; baseline (speedup 1.0000x reference)
import jax
import jax.numpy as jnp
from jax.experimental import pallas as pl


def kernel(node_attr, connectivity, edge_attr, u, We, be, Wv, bv):
    raise NotImplementedError("write your pallas kernel here")


# speedup vs baseline: 3.7523x; 3.7523x over previous
import functools

import jax
import jax.numpy as jnp
from jax import lax
from jax.experimental import pallas as pl
from jax.experimental.pallas import tpu as pltpu
from jax.experimental.pallas import tpu_sc as plsc

N = 10000
E = 320000
D = 128
DE = 16

NC = 2
NS = 16
NW = NC * NS
CHUNK = 128
NCHUNK = E // CHUNK
KMAX = -(-NCHUNK // NW)
NPAD = 10240
ROWS_PER_TILE = NPAD // NS


def _proj_body(x_ref, w_ref, ps_ref, pd_ref):
    acc = jnp.dot(x_ref[...], w_ref[...], preferred_element_type=jnp.float32)
    ps_ref[...] = acc[:, :DE]
    pd_ref[...] = acc[:, DE:]


def _node_proj(node_attr, w_sd):
    blk = 1000
    return pl.pallas_call(
        _proj_body,
        grid=(N // blk,),
        in_specs=[
            pl.BlockSpec((blk, D), lambda i: (i, 0)),
            pl.BlockSpec((D, 2 * DE), lambda i: (0, 0)),
        ],
        out_specs=[
            pl.BlockSpec((blk, DE), lambda i: (i, 0)),
            pl.BlockSpec((blk, DE), lambda i: (i, 0)),
        ],
        out_shape=[
            jax.ShapeDtypeStruct((N, DE), jnp.float32),
            jax.ShapeDtypeStruct((N, DE), jnp.float32),
        ],
    )(node_attr, w_sd)


def _pe_body(ea_ref, w_ref, b_ref, out_ref):
    out_ref[...] = (
        jnp.dot(ea_ref[...], w_ref[...], preferred_element_type=jnp.float32)
        + b_ref[...]
    )


def _edge_proj(edge_attr, w_e, be):
    blk = 3200
    return pl.pallas_call(
        _pe_body,
        grid=(E // blk,),
        in_specs=[
            pl.BlockSpec((blk, DE), lambda i: (i, 0)),
            pl.BlockSpec((DE, DE), lambda i: (0, 0)),
            pl.BlockSpec((1, DE), lambda i: (0, 0)),
        ],
        out_specs=pl.BlockSpec((blk, DE), lambda i: (i, 0)),
        out_shape=jax.ShapeDtypeStruct((E, DE), jnp.float32),
    )(edge_attr, w_e, be.reshape(1, DE))


def _sc_body(psrc, pdst, pe, row, col, eout_hbm, agg_hbm,
             ridx, cidx, src_v, dst_v, eout_v, zbuf, agg_sh, sem):
    c = lax.axis_index("c")
    s = lax.axis_index("s")
    wid = s * NC + c

    def zero_row(i, carry):
        zbuf[i, :] = jnp.zeros((DE,), jnp.float32)
        return carry

    lax.fori_loop(0, ROWS_PER_TILE, zero_row, 0)
    pltpu.sync_copy(zbuf, agg_sh.at[pl.ds(s * ROWS_PER_TILE, ROWS_PER_TILE)])
    plsc.subcore_barrier()

    def chunk(k, carry):
        cid = wid + NW * k

        @pl.when(cid < NCHUNK)
        def _():
            base = cid * CHUNK
            pltpu.sync_copy(row.at[pl.ds(base, CHUNK)], ridx)
            pltpu.sync_copy(col.at[pl.ds(base, CHUNK)], cidx)
            d1 = pltpu.async_copy(psrc.at[ridx], src_v, sem)
            d2 = pltpu.async_copy(pdst.at[cidx], dst_v, sem)
            pltpu.sync_copy(pe.at[pl.ds(base, CHUNK)], eout_v)
            d1.wait()
            d2.wait()

            def fuse_row(i, rc):
                eout_v[i, :] = jnp.maximum(
                    eout_v[i, :] + src_v[i, :] + dst_v[i, :], 0.0
                )
                return rc

            lax.fori_loop(0, CHUNK, fuse_row, 0, unroll=4)
            pltpu.sync_copy(eout_v, eout_hbm.at[pl.ds(base, CHUNK)])
            pltpu.sync_copy(eout_v, agg_sh.at[cidx], add=True)

        return carry

    lax.fori_loop(0, KMAX, chunk, 0)
    plsc.subcore_barrier()
    sl = pl.ds(s * ROWS_PER_TILE, ROWS_PER_TILE)
    pltpu.sync_copy(agg_sh.at[sl], agg_hbm.at[c].at[sl])


def _sc_edges(psrc, pdst, pe, row, col):
    mesh = plsc.VectorSubcoreMesh(core_axis_name="c", subcore_axis_name="s")
    f = pl.kernel(
        _sc_body,
        out_type=(
            jax.ShapeDtypeStruct((E, DE), jnp.float32),
            jax.ShapeDtypeStruct((NC, NPAD, DE), jnp.float32),
        ),
        mesh=mesh,
        compiler_params=pltpu.CompilerParams(use_tc_tiling_on_sc=False),
        scratch_types=[
            pltpu.VMEM((CHUNK,), jnp.int32),
            pltpu.VMEM((CHUNK,), jnp.int32),
            pltpu.VMEM((CHUNK, DE), jnp.float32),
            pltpu.VMEM((CHUNK, DE), jnp.float32),
            pltpu.VMEM((CHUNK, DE), jnp.float32),
            pltpu.VMEM((ROWS_PER_TILE, DE), jnp.float32),
            pltpu.VMEM_SHARED((NPAD, DE), jnp.float32),
            pltpu.SemaphoreType.DMA,
        ],
    )
    return f(psrc, pdst, pe, row, col)


def _vout_body(x_ref, agg_ref, w1_ref, w2_ref, b_ref, o_ref):
    a = agg_ref[0] + agg_ref[1]
    acc = jnp.dot(x_ref[...], w1_ref[...], preferred_element_type=jnp.float32)
    acc = acc + jnp.dot(a, w2_ref[...], preferred_element_type=jnp.float32)
    o_ref[...] = jnp.maximum(acc + b_ref[...], 0.0)


def _node_update(node_attr, agg, w1, w2, bv):
    blk = 1000
    return pl.pallas_call(
        _vout_body,
        grid=(N // blk,),
        in_specs=[
            pl.BlockSpec((blk, D), lambda i: (i, 0)),
            pl.BlockSpec((NC, blk, DE), lambda i: (0, i, 0)),
            pl.BlockSpec((D, D), lambda i: (0, 0)),
            pl.BlockSpec((DE, D), lambda i: (0, 0)),
            pl.BlockSpec((1, D), lambda i: (0, 0)),
        ],
        out_specs=pl.BlockSpec((blk, D), lambda i: (i, 0)),
        out_shape=jax.ShapeDtypeStruct((N, D), jnp.float32),
    )(node_attr, agg, w1, w2, bv.reshape(1, D))


def kernel(node_attr, connectivity, edge_attr, u, We, be, Wv, bv):
    row = connectivity[0]
    col = connectivity[1]
    w_sd = jnp.concatenate([We[:D], We[D:2 * D]], axis=1)
    w_e = We[2 * D:]

    psrc, pdst = _node_proj(node_attr, w_sd)
    pe = _edge_proj(edge_attr, w_e, be)
    e_out, agg = _sc_edges(psrc, pdst, pe, row, col)
    v_out = _node_update(node_attr, agg, Wv[:D], Wv[D:], bv)
    global_attr = jnp.zeros_like(u)
    return (v_out, e_out, global_attr)


# speedup vs baseline: 6.5250x; 1.7389x over previous
import functools

import jax
import jax.numpy as jnp
from jax import lax
from jax.experimental import pallas as pl
from jax.experimental.pallas import tpu as pltpu
from jax.experimental.pallas import tpu_sc as plsc

N = 10000
E = 320000
D = 128
DE = 16

NC = 2
NS = 16
NW = NC * NS
CHUNK = 125
E_PER_W = E // NW
KW = E_PER_W // CHUNK
NCHUNK = E // CHUNK
NPAD = 10240
ROWS_PER_TILE = NPAD // NS


def _proj_body(x_ref, w_ref, ps_ref, pd_ref):
    acc = jnp.dot(x_ref[...], w_ref[...], preferred_element_type=jnp.float32)
    ps_ref[...] = acc[:, :DE]
    pd_ref[...] = acc[:, DE:]


def _node_proj(node_attr, w_sd):
    blk = 1000
    return pl.pallas_call(
        _proj_body,
        grid=(N // blk,),
        in_specs=[
            pl.BlockSpec((blk, D), lambda i: (i, 0)),
            pl.BlockSpec((D, 2 * DE), lambda i: (0, 0)),
        ],
        out_specs=[
            pl.BlockSpec((blk, DE), lambda i: (i, 0)),
            pl.BlockSpec((blk, DE), lambda i: (i, 0)),
        ],
        out_shape=[
            jax.ShapeDtypeStruct((N, DE), jnp.float32),
            jax.ShapeDtypeStruct((N, DE), jnp.float32),
        ],
    )(node_attr, w_sd)


def _pe_body(ea_ref, w_ref, b_ref, out_ref):
    out_ref[...] = (
        jnp.dot(ea_ref[...], w_ref[...], preferred_element_type=jnp.float32)
        + b_ref[...]
    )


def _edge_proj(ea128, w_blk, be_tiled):
    blk = 2000
    rows = E // 8
    return pl.pallas_call(
        _pe_body,
        grid=(rows // blk,),
        in_specs=[
            pl.BlockSpec((blk, D), lambda i: (i, 0)),
            pl.BlockSpec((D, D), lambda i: (0, 0)),
            pl.BlockSpec((1, D), lambda i: (0, 0)),
        ],
        out_specs=pl.BlockSpec((blk, D), lambda i: (i, 0)),
        out_shape=jax.ShapeDtypeStruct((rows, D), jnp.float32),
    )(ea128, w_blk, be_tiled.reshape(1, D))


def _sc_body(psrc, pdst, pe1d, row2, col2, eout1d, agg_hbm,
             ridx_all, cidx_all, src_v, dst_v, pe_v, eo1_v, eo2_v, zbuf, agg_sh,
             gsem0, gsem1, psem0, psem1, osem0, osem1, ssem0, ssem1):
    c = lax.axis_index("c")
    s = lax.axis_index("s")
    wid = s * NC + c
    gsem = (gsem0, gsem1)
    psem = (psem0, psem1)
    osem = (osem0, osem1)
    ssem = (ssem0, ssem1)

    def zero_row(i, carry):
        zbuf[i, :] = jnp.zeros((DE,), jnp.float32)
        return carry

    lax.fori_loop(0, ROWS_PER_TILE, zero_row, 0)
    pltpu.sync_copy(zbuf, agg_sh.at[pl.ds(s * ROWS_PER_TILE, ROWS_PER_TILE)])

    pltpu.sync_copy(row2.at[pl.ds(wid * KW, KW)], ridx_all)
    pltpu.sync_copy(col2.at[pl.ds(wid * KW, KW)], cidx_all)
    plsc.subcore_barrier()

    ebase = wid * E_PER_W

    def issue_loads(j, b):
        pltpu.async_copy(psrc.at[ridx_all.at[j]], src_v.at[b], gsem[b])
        pltpu.async_copy(pdst.at[cidx_all.at[j]], dst_v.at[b], gsem[b])
        pltpu.async_copy(
            pe1d.at[pl.ds((ebase + j * CHUNK) * DE, CHUNK * DE)],
            pe_v.at[b], psem[b],
        )

    def wait_loads(j, b):
        pltpu.make_async_copy(psrc.at[ridx_all.at[j]], src_v.at[b], gsem[b]).wait()
        pltpu.make_async_copy(pdst.at[cidx_all.at[j]], dst_v.at[b], gsem[b]).wait()
        pltpu.make_async_copy(
            pe1d.at[pl.ds((ebase + j * CHUNK) * DE, CHUNK * DE)],
            pe_v.at[b], psem[b],
        ).wait()

    def issue_stores(j, b):
        pltpu.async_copy(
            eo1_v.at[b],
            eout1d.at[pl.ds((ebase + j * CHUNK) * DE, CHUNK * DE)],
            osem[b],
        )
        pltpu.async_copy(eo2_v.at[b], agg_sh.at[cidx_all.at[j]], ssem[b], add=True)

    def wait_stores(j, b):
        pltpu.make_async_copy(
            eo1_v.at[b],
            eout1d.at[pl.ds((ebase + j * CHUNK) * DE, CHUNK * DE)],
            osem[b],
        ).wait()
        pltpu.make_async_copy(
            eo2_v.at[b], agg_sh.at[cidx_all.at[j]], ssem[b]
        ).wait()

    issue_loads(0, 0)

    def outer(i, carry):
        for b in (0, 1):
            j = 2 * i + b

            @pl.when(j + 1 < KW)
            def _():
                issue_loads(j + 1, 1 - b)

            wait_loads(j, b)

            @pl.when(j >= 2)
            def _():
                wait_stores(j - 2, b)

            def fuse_row(r, rc):
                v = jnp.maximum(
                    pe_v[b, pl.ds(r * DE, DE)] + src_v[b, r, :] + dst_v[b, r, :],
                    0.0,
                )
                eo1_v[b, pl.ds(r * DE, DE)] = v
                eo2_v[b, r, :] = v
                return rc

            lax.fori_loop(0, CHUNK, fuse_row, 0, unroll=4)
            issue_stores(j, b)
        return carry

    lax.fori_loop(0, KW // 2, outer, 0)
    wait_stores(KW - 2, 0)
    wait_stores(KW - 1, 1)
    plsc.subcore_barrier()
    sl = pl.ds(s * ROWS_PER_TILE, ROWS_PER_TILE)
    pltpu.sync_copy(agg_sh.at[sl], agg_hbm.at[c].at[sl])


def _sc_edges(psrc, pdst, pe1d, row2, col2):
    mesh = plsc.VectorSubcoreMesh(core_axis_name="c", subcore_axis_name="s")
    f = pl.kernel(
        _sc_body,
        out_type=(
            jax.ShapeDtypeStruct((E * DE,), jnp.float32),
            jax.ShapeDtypeStruct((NC, NPAD, DE), jnp.float32),
        ),
        mesh=mesh,
        compiler_params=pltpu.CompilerParams(use_tc_tiling_on_sc=False),
        scratch_types=[
            pltpu.VMEM((KW, CHUNK), jnp.int32),
            pltpu.VMEM((KW, CHUNK), jnp.int32),
            pltpu.VMEM((2, CHUNK, DE), jnp.float32),
            pltpu.VMEM((2, CHUNK, DE), jnp.float32),
            pltpu.VMEM((2, CHUNK * DE), jnp.float32),
            pltpu.VMEM((2, CHUNK * DE), jnp.float32),
            pltpu.VMEM((2, CHUNK, DE), jnp.float32),
            pltpu.VMEM((ROWS_PER_TILE, DE), jnp.float32),
            pltpu.VMEM_SHARED((NPAD, DE), jnp.float32),
        ] + [pltpu.SemaphoreType.DMA] * 8,
    )
    return f(psrc, pdst, pe1d, row2, col2)


def _vout_body(x_ref, agg_ref, w1_ref, w2_ref, b_ref, o_ref):
    a = agg_ref[0] + agg_ref[1]
    acc = jnp.dot(x_ref[...], w1_ref[...], preferred_element_type=jnp.float32)
    acc = acc + jnp.dot(a, w2_ref[...], preferred_element_type=jnp.float32)
    o_ref[...] = jnp.maximum(acc + b_ref[...], 0.0)


def _node_update(node_attr, agg, w1, w2, bv):
    blk = 1000
    return pl.pallas_call(
        _vout_body,
        grid=(N // blk,),
        in_specs=[
            pl.BlockSpec((blk, D), lambda i: (i, 0)),
            pl.BlockSpec((NC, blk, DE), lambda i: (0, i, 0)),
            pl.BlockSpec((D, D), lambda i: (0, 0)),
            pl.BlockSpec((DE, D), lambda i: (0, 0)),
            pl.BlockSpec((1, D), lambda i: (0, 0)),
        ],
        out_specs=pl.BlockSpec((blk, D), lambda i: (i, 0)),
        out_shape=jax.ShapeDtypeStruct((N, D), jnp.float32),
    )(node_attr, agg, w1, w2, bv.reshape(1, D))


def kernel(node_attr, connectivity, edge_attr, u, We, be, Wv, bv):
    row2 = connectivity[0].reshape(NCHUNK, CHUNK)
    col2 = connectivity[1].reshape(NCHUNK, CHUNK)
    w_sd = jnp.concatenate([We[:D], We[D:2 * D]], axis=1)
    w_blk = jnp.kron(jnp.eye(8, dtype=jnp.float32), We[2 * D:])
    be_tiled = jnp.tile(be, 8)

    psrc, pdst = _node_proj(node_attr, w_sd)
    pe128 = _edge_proj(edge_attr.reshape(E // 8, D), w_blk, be_tiled)
    e_out1d, agg = _sc_edges(psrc, pdst, pe128.reshape(E * DE), row2, col2)
    v_out = _node_update(node_attr, agg, Wv[:D], Wv[D:], bv)
    global_attr = jnp.zeros_like(u)
    return (v_out, e_out1d.reshape(E, DE), global_attr)
